# baseline (speedup 1.0000x reference)
import functools

import jax
import jax.numpy as jnp
from jax import lax
from jax.experimental import pallas as pl
from jax.experimental.pallas import tpu as pltpu
from jax.experimental.pallas import tpu_sc as plsc

_VOCAB = 1000000
_D = 32
_B = 4096
_NC = 2
_NS = 16
_NW = _NC * _NS
_BPW = _B // _NW
_K = 32
_NCHUNK = _BPW // _K

_ROW_BLOCK = 512


@functools.partial(
    pl.kernel,
    out_type=(
        jax.ShapeDtypeStruct((_B, _D), jnp.float32),
        jax.ShapeDtypeStruct((_B, _D), jnp.float32),
    ),
    mesh=plsc.VectorSubcoreMesh(core_axis_name="c", subcore_axis_name="s"),
    compiler_params=pltpu.CompilerParams(needs_layout_passes=False),
    scratch_types=(
        pltpu.VMEM((_BPW,), jnp.int32),
        pltpu.VMEM((_K, 8, _D), jnp.float32),
        pltpu.VMEM((_K, 8, _D), jnp.float32),
        pltpu.VMEM((_BPW, _D), jnp.float32),
        pltpu.SemaphoreType.DMA,
        pltpu.SemaphoreType.DMA,
    ),
)
def _gather_sc(emb_hbm, tgt_hbm, ctx_hbm, out_t_hbm, out_c_hbm,
               idx_v, raw_a, raw_b, out_v, sem_a, sem_b):
    wid = lax.axis_index("s") * _NC + lax.axis_index("c")
    base = wid * _BPW
    bufs = (raw_a, raw_b)
    sems = (sem_a, sem_b)

    for idx_hbm, out_hbm in ((tgt_hbm, out_t_hbm), (ctx_hbm, out_c_hbm)):
        pltpu.sync_copy(idx_hbm.at[pl.ds(base, _BPW)], idx_v)

        def _fire(k):
            buf, sem = bufs[k % 2], sems[k % 2]
            cps = []
            for g in range(_K // 16):
                vec = idx_v[pl.ds(k * _K + g * 16, 16)]
                for l in range(16):
                    v = vec[l]
                    cps.append(pltpu.async_copy(
                        emb_hbm.at[pl.ds((v >> 3) * 8, 8)],
                        buf.at[g * 16 + l], sem))
            return cps

        pending = _fire(0)
        for k in range(_NCHUNK):
            nxt = _fire(k + 1) if k + 1 < _NCHUNK else []
            for cp in pending:
                cp.wait()
            pending = nxt
            raw = bufs[k % 2]

            def _extract(t, _, k=k, raw=raw):
                w = lax.iota(jnp.int32, 16) + (t + k * (_K * _D // 16)) * 16
                j = w >> 5
                c = w & 31
                s = plsc.load_gather(idx_v, [j]) & 7
                val = plsc.load_gather(raw, [j - k * _K, s, c])
                plsc.store_scatter(out_v, [j, c], val)
                return 0

            lax.fori_loop(0, _K * _D // 16, _extract, 0, unroll=4)

        pltpu.sync_copy(out_v, out_hbm.at[pl.ds(base, _BPW)])


def _scores_body(t_ref, c_ref, o_ref):
    o_ref[...] = lax.dot_general(
        t_ref[...], c_ref[...],
        dimension_numbers=(((1,), (1,)), ((), ())),
        preferred_element_type=jnp.float32,
    )


_scores_tc = pl.pallas_call(
    _scores_body,
    grid=(_B // _ROW_BLOCK,),
    in_specs=[
        pl.BlockSpec((_ROW_BLOCK, _D), lambda i: (i, 0)),
        pl.BlockSpec((_B, _D), lambda i: (0, 0)),
    ],
    out_specs=pl.BlockSpec((_ROW_BLOCK, _B), lambda i: (i, 0)),
    out_shape=jax.ShapeDtypeStruct((_B, _B), jnp.float32),
)


def kernel(target, context, embeddings):
    tgt_rows, ctx_rows = _gather_sc(
        embeddings, target.astype(jnp.int32), context.astype(jnp.int32))
    return _scores_tc(tgt_rows, ctx_rows)
